# baseline (speedup 1.0000x reference)
import jax
import jax.numpy as jnp
from jax.experimental import pallas as pl
from jax.experimental.pallas import tpu as pltpu

B = 8
C = 256
HW = 1024
K = 1024
N_TOTAL = B * C * HW


def _vq_kernel(z_ref, em2_hbm, q_ref, idx_ref, loss_ref, perp_ref,
               counts_ref, loss_acc, em2_vmem, dma_sem):
    b = pl.program_id(0)

    @pl.when(b == 0)
    def _load_codebook():
        cp = pltpu.make_async_copy(em2_hbm, em2_vmem, dma_sem)
        cp.start()
        cp.wait()

    em2 = em2_vmem[...]
    e_sq = jnp.sum(em2 * em2, axis=1, keepdims=True) * 0.25

    CH = 4
    W = HW // CH
    iota_k = jax.lax.broadcasted_iota(jnp.int32, (K, W), 0)
    sq = jnp.zeros((1, 1), jnp.float32)
    cnt = jnp.zeros((K, 1), jnp.float32)
    for j in range(CH):
        zc = z_ref[0, :, j * W:(j + 1) * W]
        m2 = jnp.dot(em2, zc, preferred_element_type=jnp.float32)
        z_sq = jnp.sum(zc * zc, axis=0, keepdims=True)
        dist = (z_sq + e_sq) + m2

        mn = jnp.min(dist, axis=0, keepdims=True)
        idx2 = jnp.min(jnp.where(dist == mn, iota_k, jnp.int32(K)),
                       axis=0, keepdims=True)
        idx_ref[0, :, j * W:(j + 1) * W] = idx2

        onehot = jnp.where(iota_k == idx2, jnp.float32(-0.5),
                           jnp.float32(0.0))
        q = jax.lax.dot_general(em2, onehot,
                                (((0,), (0,)), ((), ())),
                                preferred_element_type=jnp.float32)
        q_ref[0, :, j * W:(j + 1) * W] = q

        sq = sq + jnp.sum(mn, keepdims=True).reshape(1, 1)
        cnt = cnt + jnp.sum(onehot, axis=1, keepdims=True) * (-2.0)

    @pl.when(b == 0)
    def _init():
        loss_acc[...] = sq
        counts_ref[...] = cnt

    @pl.when(b != 0)
    def _accum():
        loss_acc[...] = loss_acc[...] + sq
        counts_ref[...] = counts_ref[...] + cnt

    @pl.when(b == B - 1)
    def _finalize():
        mse = loss_acc[...] * jnp.float32(1.0 / N_TOTAL)
        loss_ref[...] = mse + jnp.float32(0.25) * mse
        p = counts_ref[...] * jnp.float32(1.0 / (B * HW))
        ent = -jnp.sum(p * jnp.log(p + jnp.float32(1e-10)),
                       keepdims=True).reshape(1, 1)
        perp_ref[...] = jnp.exp(ent)


def kernel(z, embedding):
    z_r = z.reshape(B, C, HW)
    q, idx, loss, perp = pl.pallas_call(
        _vq_kernel,
        grid=(B,),
        in_specs=[
            pl.BlockSpec((1, C, HW), lambda b: (b, 0, 0)),
            pl.BlockSpec(memory_space=pltpu.MemorySpace.HBM),
        ],
        out_specs=[
            pl.BlockSpec((1, C, HW), lambda b: (b, 0, 0)),
            pl.BlockSpec((1, 1, HW), lambda b: (b, 0, 0)),
            pl.BlockSpec((1, 1), lambda b: (0, 0)),
            pl.BlockSpec((1, 1), lambda b: (0, 0)),
        ],
        out_shape=[
            jax.ShapeDtypeStruct((B, C, HW), jnp.float32),
            jax.ShapeDtypeStruct((B, 1, HW), jnp.int32),
            jax.ShapeDtypeStruct((1, 1), jnp.float32),
            jax.ShapeDtypeStruct((1, 1), jnp.float32),
        ],
        scratch_shapes=[
            pltpu.VMEM((K, 1), jnp.float32),
            pltpu.VMEM((1, 1), jnp.float32),
            pltpu.VMEM((K, C), jnp.float32),
            pltpu.SemaphoreType.DMA,
        ],
        compiler_params=pltpu.CompilerParams(
            dimension_semantics=("arbitrary",)),
    )(z_r, embedding * jnp.float32(-2.0))
    return (q.reshape(z.shape), loss[0, 0], perp[0, 0], idx.reshape(-1))


# speedup vs baseline: 1.0623x; 1.0623x over previous
import jax
import jax.numpy as jnp
from jax.experimental import pallas as pl
from jax.experimental.pallas import tpu as pltpu

B = 8
C = 256
HW = 1024
K = 1024
N_TOTAL = B * C * HW


def _vq_kernel(z_ref, em2_hbm, q_ref, idx_ref, loss_ref, perp_ref,
               counts_ref, loss_acc, em2_vmem, dma_sem):
    b = pl.program_id(0)

    @pl.when(b == 0)
    def _load_codebook():
        cp = pltpu.make_async_copy(em2_hbm, em2_vmem, dma_sem)
        cp.start()
        cp.wait()

    em2 = em2_vmem[...]
    e_sq = jnp.sum(em2 * em2, axis=1, keepdims=True) * 0.25

    CH = 4
    W = HW // CH
    iota_k = jax.lax.broadcasted_iota(jnp.int32, (K, W), 0)
    sq = jnp.zeros((1, 1), jnp.float32)
    cnt = jnp.zeros((K, 1), jnp.float32)
    for j in range(CH):
        zc = z_ref[0, :, j * W:(j + 1) * W]
        m2 = jnp.dot(em2, zc, preferred_element_type=jnp.float32)
        z_sq = jnp.sum(zc * zc, axis=0, keepdims=True)
        dist = (z_sq + e_sq) + m2

        idx2 = jnp.argmin(dist, axis=0)[None, :]
        idx_ref[0, :, j * W:(j + 1) * W] = idx2

        onehot = jnp.where(iota_k == idx2, jnp.float32(-0.5),
                           jnp.float32(0.0))
        q = jax.lax.dot_general(em2, onehot,
                                (((0,), (0,)), ((), ())),
                                preferred_element_type=jnp.float32)
        q_ref[0, :, j * W:(j + 1) * W] = q

        d = q - zc
        sq = sq + jnp.sum(d * d, keepdims=True).reshape(1, 1)
        cnt = cnt + jnp.sum(onehot, axis=1, keepdims=True) * (-2.0)

    @pl.when(b == 0)
    def _init():
        loss_acc[...] = sq
        counts_ref[...] = cnt

    @pl.when(b != 0)
    def _accum():
        loss_acc[...] = loss_acc[...] + sq
        counts_ref[...] = counts_ref[...] + cnt

    @pl.when(b == B - 1)
    def _finalize():
        mse = loss_acc[...] * jnp.float32(1.0 / N_TOTAL)
        loss_ref[...] = mse + jnp.float32(0.25) * mse
        p = counts_ref[...] * jnp.float32(1.0 / (B * HW))
        ent = -jnp.sum(p * jnp.log(p + jnp.float32(1e-10)),
                       keepdims=True).reshape(1, 1)
        perp_ref[...] = jnp.exp(ent)


def kernel(z, embedding):
    z_r = z.reshape(B, C, HW)
    q, idx, loss, perp = pl.pallas_call(
        _vq_kernel,
        grid=(B,),
        in_specs=[
            pl.BlockSpec((1, C, HW), lambda b: (b, 0, 0)),
            pl.BlockSpec(memory_space=pltpu.MemorySpace.HBM),
        ],
        out_specs=[
            pl.BlockSpec((1, C, HW), lambda b: (b, 0, 0)),
            pl.BlockSpec((1, 1, HW), lambda b: (b, 0, 0)),
            pl.BlockSpec((1, 1), lambda b: (0, 0)),
            pl.BlockSpec((1, 1), lambda b: (0, 0)),
        ],
        out_shape=[
            jax.ShapeDtypeStruct((B, C, HW), jnp.float32),
            jax.ShapeDtypeStruct((B, 1, HW), jnp.int32),
            jax.ShapeDtypeStruct((1, 1), jnp.float32),
            jax.ShapeDtypeStruct((1, 1), jnp.float32),
        ],
        scratch_shapes=[
            pltpu.VMEM((K, 1), jnp.float32),
            pltpu.VMEM((1, 1), jnp.float32),
            pltpu.VMEM((K, C), jnp.float32),
            pltpu.SemaphoreType.DMA,
        ],
        compiler_params=pltpu.CompilerParams(
            dimension_semantics=("arbitrary",)),
    )(z_r, embedding * jnp.float32(-2.0))
    return (q.reshape(z.shape), loss[0, 0], perp[0, 0], idx.reshape(-1))
